# baseline (speedup 1.0000x reference)
import functools

import jax
import jax.numpy as jnp
from jax import lax
from jax.experimental import pallas as pl
from jax.experimental.pallas import tpu as pltpu
from jax.experimental.pallas import tpu_sc as plsc

SEQ_LEN = 8192
D_MODEL = 1024
LANES = 16
NUM_WORKERS = 32
ROWS_PER_WORKER = SEQ_LEN // NUM_WORKERS
CHUNK_ROWS = 8
NUM_CHUNKS = ROWS_PER_WORKER // CHUNK_ROWS
NBUF = 4

_mesh = plsc.VectorSubcoreMesh(core_axis_name="c", subcore_axis_name="s")

_CHUNK = (CHUNK_ROWS, D_MODEL)
_scratch = (
    [pltpu.VMEM(_CHUNK, jnp.float32) for _ in range(3 * NBUF)]
    + [pltpu.SemaphoreType.DMA for _ in range(3 * NBUF)]
)


@functools.partial(
    pl.kernel,
    mesh=_mesh,
    out_type=jax.ShapeDtypeStruct((SEQ_LEN, D_MODEL), jnp.float32),
    scratch_types=_scratch,
)
def _sc_add(x_hbm, emb_hbm, out_hbm, *scratch):
    bufs = scratch[: 3 * NBUF]
    sems = scratch[3 * NBUF :]
    xbufs, ebufs, obufs = bufs[:NBUF], bufs[NBUF : 2 * NBUF], bufs[2 * NBUF :]
    sxs, ses, sos = sems[:NBUF], sems[NBUF : 2 * NBUF], sems[2 * NBUF :]

    wid = lax.axis_index("s") * 2 + lax.axis_index("c")
    base = wid * ROWS_PER_WORKER

    def rows_at(ci):
        return pl.ds(base + ci * CHUNK_ROWS, CHUNK_ROWS)

    def start_gather(ci, b):
        pltpu.async_copy(x_hbm.at[rows_at(ci), :], xbufs[b], sxs[b])
        pltpu.async_copy(emb_hbm.at[rows_at(ci), :], ebufs[b], ses[b])

    def wait_gather(b):
        pltpu.make_async_copy(x_hbm.at[rows_at(0), :], xbufs[b], sxs[b]).wait()
        pltpu.make_async_copy(emb_hbm.at[rows_at(0), :], ebufs[b], ses[b]).wait()

    def wait_scatter(b):
        pltpu.make_async_copy(obufs[b], out_hbm.at[rows_at(0), :], sos[b]).wait()

    for b in range(NBUF - 1):
        start_gather(b, b)

    def outer(g, carry):
        for b in range(NBUF):
            ci = NBUF * g + b

            @pl.when(ci + NBUF - 1 < NUM_CHUNKS)
            def _():
                start_gather(ci + NBUF - 1, (b + NBUF - 1) % NBUF)

            wait_gather(b)

            @pl.when(ci >= NBUF)
            def _():
                wait_scatter(b)

            xbuf, ebuf, obuf = xbufs[b], ebufs[b], obufs[b]

            def row_body(r, rcarry):
                for j in range(D_MODEL // LANES):
                    sl = pl.ds(j * LANES, LANES)
                    obuf[r, sl] = xbuf[r, sl] + ebuf[r, sl]
                return rcarry

            lax.fori_loop(0, CHUNK_ROWS, row_body, 0)
            pltpu.async_copy(obuf, out_hbm.at[rows_at(ci), :], sos[b])
        return carry

    lax.fori_loop(0, NUM_CHUNKS // NBUF, outer, 0)
    for b in range(NBUF):
        wait_scatter(b)


def kernel(x, emb):
    return _sc_add(x, emb)
